# baseline (speedup 1.0000x reference)
import functools
import jax
import jax.numpy as jnp
from jax import lax
from jax.experimental import pallas as pl
from jax.experimental.pallas import tpu as pltpu
from jax.experimental.pallas import tpu_sc as plsc

D = 128
LANES = 16
NV = D // LANES
NC = 2
NS = 16
NW = NC * NS


def _make_stage(nrows, nseg_pad, spw, s_blk, r_chunk, depth=2):
    assert nseg_pad == NW * spw and spw % s_blk == 0 and s_blk % 8 == 0
    assert nrows % r_chunk == 0 and (r_chunk & (r_chunk - 1)) == 0
    assert depth & (depth - 1) == 0
    nblk = spw // s_blk
    mesh = plsc.VectorSubcoreMesh(core_axis_name="c", subcore_axis_name="s")

    @functools.partial(
        pl.kernel,
        mesh=mesh,
        out_type=jax.ShapeDtypeStruct((nseg_pad, D), jnp.float32),
        scratch_types=[
            pltpu.VMEM((s_blk + LANES,), jnp.int32),
            pltpu.VMEM((depth * r_chunk, D), jnp.float32),
            pltpu.VMEM((2 * s_blk, D), jnp.float32),
            pltpu.SemaphoreType.DMA,
            pltpu.SemaphoreType.DMA,
        ]
        + [pltpu.SemaphoreType.DMA] * depth,
    )
    def stage(x_hbm, offs_hbm, out_hbm, offs_v, buf_v, out_v, sem, sem_o, *half_sems):
        wid = lax.axis_index("s") * NC + lax.axis_index("c")
        neg = jnp.full((LANES,), -jnp.inf, jnp.float32)
        zeros = jnp.zeros((LANES,), jnp.float32)
        cint = jnp.int32(r_chunk)

        def issue_half(base, p):
            base = pl.multiple_of(base, r_chunk)
            pltpu.async_copy(
                x_hbm.at[pl.ds(base, r_chunk)],
                buf_v.at[pl.ds(p * r_chunk, r_chunk)],
                half_sems[p],
            )

        def wait_half(p):
            pltpu.make_async_copy(
                x_hbm.at[pl.ds(0, r_chunk)],
                buf_v.at[pl.ds(p * r_chunk, r_chunk)],
                half_sems[p],
            ).wait()

        def by_parity(base, fn):
            slot = (base // cint) & (depth - 1)
            for p in range(depth):
                @pl.when(slot == p)
                def _(p=p):
                    fn(p)

        def refill(r):
            by_parity(r, wait_half)
            nb = r + jnp.int32((depth - 1) * r_chunk)

            @pl.when(nb < jnp.int32(nrows))
            def _():
                by_parity(nb, lambda p: issue_half(nb, p))

        pltpu.async_copy(
            offs_hbm.at[pl.ds(wid * spw, LANES)], offs_v.at[pl.ds(0, LANES)], sem
        ).wait()
        rbeg = offs_v[pl.ds(0, LANES)][0]
        b0 = jnp.minimum((rbeg // cint) * cint, jnp.int32(nrows - r_chunk))
        by_parity(b0, lambda p: issue_half(b0, p))
        by_parity(b0, wait_half)
        hi0 = b0 + cint
        for k in range(1, depth):
            bk = b0 + jnp.int32(k * r_chunk)

            @pl.when(bk < jnp.int32(nrows))
            def _(bk=bk):
                by_parity(bk, lambda p: issue_half(bk, p))

        ring_mask = jnp.int32(depth * r_chunk - 1)

        def blk_body(blk, carry):
            seg0 = wid * spw + blk * s_blk
            obase = jnp.bitwise_and(blk, 1) * jnp.int32(s_blk)
            pltpu.async_copy(offs_hbm.at[pl.ds(seg0, s_blk + LANES)], offs_v, sem).wait()

            @pl.when(blk > 0)
            def _():
                pltpu.make_async_copy(
                    out_v.at[pl.ds(0, s_blk)], out_hbm.at[pl.ds(0, s_blk)], sem_o
                ).wait()

            def seg_body(j, cst):
                hi_s, beg = cst
                end = offs_v[pl.ds(j, LANES)][1]

                ne = end > beg

                def store(acc):
                    for i in range(NV):
                        out_v[obase + j, pl.ds(i * LANES, LANES)] = jnp.where(
                            ne, acc[i], zeros
                        )

                def fast(_):
                    base = jnp.bitwise_and(beg, ring_mask)
                    lim = base + (end - beg)

                    @pl.loop(base, lim, init_carry=(neg,) * NV, step=2)
                    def facc(k, acc):
                        k2 = jnp.minimum(k + 1, lim - 1)
                        a = [
                            jnp.maximum(acc[i], buf_v[k, pl.ds(i * LANES, LANES)])
                            for i in range(NV)
                        ]
                        return tuple(
                            jnp.maximum(a[i], buf_v[k2, pl.ds(i * LANES, LANES)])
                            for i in range(NV)
                        )

                    store(facc)
                    return hi_s

                def slow(_):
                    def row_body(r, rst):
                        @pl.when(r >= rst[0])
                        def _():
                            refill(r)

                        hi = jnp.where(r >= rst[0], r + cint, rst[0])
                        row = jnp.bitwise_and(r, ring_mask)
                        acc = [
                            jnp.maximum(rst[1 + i], buf_v[row, pl.ds(i * LANES, LANES)])
                            for i in range(NV)
                        ]
                        return (hi, *acc)

                    rst = lax.fori_loop(beg, end, row_body, (hi_s,) + (neg,) * NV)
                    store(list(rst[1:]))
                    return rst[0]

                hi_n = lax.cond(end <= hi_s, fast, slow, 0)
                return (hi_n, end)

            carry = lax.fori_loop(0, s_blk, seg_body, carry)
            pltpu.async_copy(
                out_v.at[pl.ds(obase, s_blk)], out_hbm.at[pl.ds(seg0, s_blk)], sem_o
            )
            return carry

        hi_end, _ = lax.fori_loop(0, nblk, blk_body, (hi0, rbeg))

        pltpu.make_async_copy(
            out_v.at[pl.ds(0, s_blk)], out_hbm.at[pl.ds(0, s_blk)], sem_o
        ).wait()

        for k in range(max(1, depth - 1)):
            bk = hi_end + jnp.int32(k * r_chunk)

            @pl.when(bk < jnp.int32(nrows))
            def _(bk=bk):
                by_parity(bk, wait_half)


    return stage


_stage1 = _make_stage(
    nrows=320000, nseg_pad=64000, spw=2000, s_blk=200, r_chunk=128, depth=4
)
_stage2 = _make_stage(
    nrows=64000, nseg_pad=10240, spw=320, s_blk=160, r_chunk=128, depth=4
)


def kernel(x_main, x_mod, csr_idx_unit, csr_idx_view):
    del x_main
    offs1 = jnp.concatenate(
        [csr_idx_unit, jnp.broadcast_to(csr_idx_unit[-1:], (23,))]
    )
    offs2 = jnp.concatenate(
        [csr_idx_view, jnp.full((10256 - 10001,), 64000, jnp.int32)]
    )
    x_agg = _stage1(x_mod, offs1)
    out = _stage2(x_agg, offs2)
    return out[:10000]


# speedup vs baseline: 1.0293x; 1.0293x over previous
import functools
import jax
import jax.numpy as jnp
from jax import lax
from jax.experimental import pallas as pl
from jax.experimental.pallas import tpu as pltpu
from jax.experimental.pallas import tpu_sc as plsc

D = 128
LANES = 16
NV = D // LANES
NC = 2
NS = 16
NW = NC * NS


def _make_stage(nrows, nseg_pad, spw, s_blk, r_chunk, depth=2):
    assert nseg_pad == NW * spw and spw % s_blk == 0 and s_blk % 8 == 0
    assert nrows % r_chunk == 0 and (r_chunk & (r_chunk - 1)) == 0
    assert depth & (depth - 1) == 0
    nblk = spw // s_blk
    mesh = plsc.VectorSubcoreMesh(core_axis_name="c", subcore_axis_name="s")

    @functools.partial(
        pl.kernel,
        mesh=mesh,
        out_type=jax.ShapeDtypeStruct((nseg_pad, D), jnp.float32),
        scratch_types=[
            pltpu.VMEM((s_blk + LANES,), jnp.int32),
            pltpu.VMEM((depth * r_chunk, D), jnp.float32),
            pltpu.VMEM((2 * s_blk, D), jnp.float32),
            pltpu.SemaphoreType.DMA,
            pltpu.SemaphoreType.DMA,
        ]
        + [pltpu.SemaphoreType.DMA] * depth,
    )
    def stage(x_hbm, offs_hbm, out_hbm, offs_v, buf_v, out_v, sem, sem_o, *half_sems):
        wid = lax.axis_index("s") * NC + lax.axis_index("c")
        neg = jnp.full((LANES,), -jnp.inf, jnp.float32)
        zeros = jnp.zeros((LANES,), jnp.float32)
        cint = jnp.int32(r_chunk)

        def issue_half(base, p):
            base = pl.multiple_of(base, r_chunk)
            pltpu.async_copy(
                x_hbm.at[pl.ds(base, r_chunk)],
                buf_v.at[pl.ds(p * r_chunk, r_chunk)],
                half_sems[p],
            )

        def wait_half(p):
            pltpu.make_async_copy(
                x_hbm.at[pl.ds(0, r_chunk)],
                buf_v.at[pl.ds(p * r_chunk, r_chunk)],
                half_sems[p],
            ).wait()

        def by_parity(base, fn):
            slot = (base // cint) & (depth - 1)
            for p in range(depth):
                @pl.when(slot == p)
                def _(p=p):
                    fn(p)

        def refill(r):
            by_parity(r, wait_half)
            nb = r + jnp.int32((depth - 1) * r_chunk)

            @pl.when(nb < jnp.int32(nrows))
            def _():
                by_parity(nb, lambda p: issue_half(nb, p))

        pltpu.async_copy(
            offs_hbm.at[pl.ds(wid * spw, LANES)], offs_v.at[pl.ds(0, LANES)], sem
        ).wait()
        rbeg = offs_v[pl.ds(0, LANES)][0]
        b0 = jnp.minimum((rbeg // cint) * cint, jnp.int32(nrows - r_chunk))
        by_parity(b0, lambda p: issue_half(b0, p))
        by_parity(b0, wait_half)
        hi0 = b0 + cint
        for k in range(1, depth):
            bk = b0 + jnp.int32(k * r_chunk)

            @pl.when(bk < jnp.int32(nrows))
            def _(bk=bk):
                by_parity(bk, lambda p: issue_half(bk, p))

        ring_mask = jnp.int32(depth * r_chunk - 1)

        def blk_body(blk, carry):
            seg0 = wid * spw + blk * s_blk
            obase = jnp.bitwise_and(blk, 1) * jnp.int32(s_blk)
            pltpu.async_copy(offs_hbm.at[pl.ds(seg0, s_blk + LANES)], offs_v, sem).wait()

            @pl.when(blk > 0)
            def _():
                pltpu.make_async_copy(
                    out_v.at[pl.ds(0, s_blk)], out_hbm.at[pl.ds(0, s_blk)], sem_o
                ).wait()

            def seg_body(j, cst):
                hi_s, beg, end = cst
                end_nx = offs_v[pl.ds(j + 1, LANES)][1]

                ne = end > beg

                def store(acc):
                    for i in range(NV):
                        out_v[obase + j, pl.ds(i * LANES, LANES)] = jnp.where(
                            ne, acc[i], zeros
                        )

                def fast(_):
                    base = jnp.bitwise_and(beg, ring_mask)
                    lim = base + (end - beg)

                    @pl.loop(base, lim, init_carry=(neg,) * NV, step=2)
                    def facc(k, acc):
                        k2 = jnp.minimum(k + 1, lim - 1)
                        a = [
                            jnp.maximum(acc[i], buf_v[k, pl.ds(i * LANES, LANES)])
                            for i in range(NV)
                        ]
                        return tuple(
                            jnp.maximum(a[i], buf_v[k2, pl.ds(i * LANES, LANES)])
                            for i in range(NV)
                        )

                    store(facc)
                    return hi_s

                def slow(_):
                    def row_body(r, rst):
                        @pl.when(r >= rst[0])
                        def _():
                            refill(r)

                        hi = jnp.where(r >= rst[0], r + cint, rst[0])
                        row = jnp.bitwise_and(r, ring_mask)
                        acc = [
                            jnp.maximum(rst[1 + i], buf_v[row, pl.ds(i * LANES, LANES)])
                            for i in range(NV)
                        ]
                        return (hi, *acc)

                    rst = lax.fori_loop(beg, end, row_body, (hi_s,) + (neg,) * NV)
                    store(list(rst[1:]))
                    return rst[0]

                hi_n = lax.cond(end <= hi_s, fast, slow, 0)
                return (hi_n, end, end_nx)

            end0 = offs_v[pl.ds(0, LANES)][1]
            st = lax.fori_loop(0, s_blk, seg_body, (carry[0], carry[1], end0))
            carry = (st[0], st[1])
            pltpu.async_copy(
                out_v.at[pl.ds(obase, s_blk)], out_hbm.at[pl.ds(seg0, s_blk)], sem_o
            )
            return carry

        hi_end, _ = lax.fori_loop(0, nblk, blk_body, (hi0, rbeg))

        pltpu.make_async_copy(
            out_v.at[pl.ds(0, s_blk)], out_hbm.at[pl.ds(0, s_blk)], sem_o
        ).wait()

        for k in range(max(1, depth - 1)):
            bk = hi_end + jnp.int32(k * r_chunk)

            @pl.when(bk < jnp.int32(nrows))
            def _(bk=bk):
                by_parity(bk, wait_half)


    return stage


_stage1 = _make_stage(
    nrows=320000, nseg_pad=64000, spw=2000, s_blk=200, r_chunk=256, depth=2
)
_stage2 = _make_stage(
    nrows=64000, nseg_pad=10240, spw=320, s_blk=160, r_chunk=256, depth=2
)


def kernel(x_main, x_mod, csr_idx_unit, csr_idx_view):
    del x_main
    offs1 = jnp.concatenate(
        [csr_idx_unit, jnp.broadcast_to(csr_idx_unit[-1:], (23,))]
    )
    offs2 = jnp.concatenate(
        [csr_idx_view, jnp.full((10256 - 10001,), 64000, jnp.int32)]
    )
    x_agg = _stage1(x_mod, offs1)
    out = _stage2(x_agg, offs2)
    return out[:10000]


# speedup vs baseline: 1.0581x; 1.0281x over previous
import functools
import jax
import jax.numpy as jnp
from jax import lax
from jax.experimental import pallas as pl
from jax.experimental.pallas import tpu as pltpu
from jax.experimental.pallas import tpu_sc as plsc

D = 128
LANES = 16
NV = D // LANES
NC = 2
NS = 16
NW = NC * NS


def _make_stage(nrows, nseg_pad, spw, s_blk, r_chunk, depth=2):
    assert nseg_pad == NW * spw and spw % s_blk == 0 and s_blk % 8 == 0
    assert nrows % r_chunk == 0 and (r_chunk & (r_chunk - 1)) == 0
    assert depth & (depth - 1) == 0
    nblk = spw // s_blk
    mesh = plsc.VectorSubcoreMesh(core_axis_name="c", subcore_axis_name="s")

    @functools.partial(
        pl.kernel,
        mesh=mesh,
        out_type=jax.ShapeDtypeStruct((nseg_pad, D), jnp.float32),
        scratch_types=[
            pltpu.VMEM((s_blk + LANES,), jnp.int32),
            pltpu.VMEM((depth * r_chunk, D), jnp.float32),
            pltpu.VMEM((2 * s_blk, D), jnp.float32),
            pltpu.SemaphoreType.DMA,
            pltpu.SemaphoreType.DMA,
        ]
        + [pltpu.SemaphoreType.DMA] * depth,
    )
    def stage(x_hbm, offs_hbm, out_hbm, offs_v, buf_v, out_v, sem, sem_o, *half_sems):
        wid = lax.axis_index("s") * NC + lax.axis_index("c")
        neg = jnp.full((LANES,), -jnp.inf, jnp.float32)
        zeros = jnp.zeros((LANES,), jnp.float32)
        cint = jnp.int32(r_chunk)

        def issue_half(base, p):
            base = pl.multiple_of(base, r_chunk)
            pltpu.async_copy(
                x_hbm.at[pl.ds(base, r_chunk)],
                buf_v.at[pl.ds(p * r_chunk, r_chunk)],
                half_sems[p],
            )

        def wait_half(p):
            pltpu.make_async_copy(
                x_hbm.at[pl.ds(0, r_chunk)],
                buf_v.at[pl.ds(p * r_chunk, r_chunk)],
                half_sems[p],
            ).wait()

        def by_parity(base, fn):
            slot = (base // cint) & (depth - 1)
            for p in range(depth):
                @pl.when(slot == p)
                def _(p=p):
                    fn(p)

        def refill(r):
            by_parity(r, wait_half)
            nb = r + jnp.int32((depth - 1) * r_chunk)

            @pl.when(nb < jnp.int32(nrows))
            def _():
                by_parity(nb, lambda p: issue_half(nb, p))

        pltpu.async_copy(
            offs_hbm.at[pl.ds(wid * spw, LANES)], offs_v.at[pl.ds(0, LANES)], sem
        ).wait()
        rbeg = offs_v[pl.ds(0, LANES)][0]
        b0 = jnp.minimum((rbeg // cint) * cint, jnp.int32(nrows - r_chunk))
        by_parity(b0, lambda p: issue_half(b0, p))
        by_parity(b0, wait_half)
        hi0 = b0 + cint
        for k in range(1, depth):
            bk = b0 + jnp.int32(k * r_chunk)

            @pl.when(bk < jnp.int32(nrows))
            def _(bk=bk):
                by_parity(bk, lambda p: issue_half(bk, p))

        ring_mask = jnp.int32(depth * r_chunk - 1)

        def blk_body(blk, carry):
            seg0 = wid * spw + blk * s_blk
            obase = jnp.bitwise_and(blk, 1) * jnp.int32(s_blk)
            pltpu.async_copy(offs_hbm.at[pl.ds(seg0, s_blk + LANES)], offs_v, sem).wait()

            @pl.when(blk > 0)
            def _():
                pltpu.make_async_copy(
                    out_v.at[pl.ds(0, s_blk)], out_hbm.at[pl.ds(0, s_blk)], sem_o
                ).wait()

            def seg_body(j, cst):
                hi_s, beg, end = cst
                end_nx = offs_v[pl.ds(j + 1, LANES)][1]

                ne = end > beg

                def store(acc):
                    for i in range(NV):
                        out_v[obase + j, pl.ds(i * LANES, LANES)] = jnp.where(
                            ne, acc[i], zeros
                        )

                def fast(_):
                    base = jnp.bitwise_and(beg, ring_mask)
                    lim = base + (end - beg)

                    @pl.loop(base, lim, init_carry=(neg,) * NV, step=2)
                    def facc(k, acc):
                        k2 = jnp.minimum(k + 1, lim - 1)
                        a = [
                            jnp.maximum(acc[i], buf_v[k, pl.ds(i * LANES, LANES)])
                            for i in range(NV)
                        ]
                        return tuple(
                            jnp.maximum(a[i], buf_v[k2, pl.ds(i * LANES, LANES)])
                            for i in range(NV)
                        )

                    store(facc)
                    return hi_s

                def slow(_):
                    def row_body(r, rst):
                        @pl.when(r >= rst[0])
                        def _():
                            refill(r)

                        hi = jnp.where(r >= rst[0], r + cint, rst[0])
                        row = jnp.bitwise_and(r, ring_mask)
                        acc = [
                            jnp.maximum(rst[1 + i], buf_v[row, pl.ds(i * LANES, LANES)])
                            for i in range(NV)
                        ]
                        return (hi, *acc)

                    rst = lax.fori_loop(beg, end, row_body, (hi_s,) + (neg,) * NV)
                    store(list(rst[1:]))
                    return rst[0]

                hi_n = lax.cond(end <= hi_s, fast, slow, 0)
                return (hi_n, end, end_nx)

            end0 = offs_v[pl.ds(0, LANES)][1]
            st = lax.fori_loop(
                0, s_blk, seg_body, (carry[0], carry[1], end0), unroll=2
            )
            carry = (st[0], st[1])
            pltpu.async_copy(
                out_v.at[pl.ds(obase, s_blk)], out_hbm.at[pl.ds(seg0, s_blk)], sem_o
            )
            return carry

        hi_end, _ = lax.fori_loop(0, nblk, blk_body, (hi0, rbeg))

        pltpu.make_async_copy(
            out_v.at[pl.ds(0, s_blk)], out_hbm.at[pl.ds(0, s_blk)], sem_o
        ).wait()

        for k in range(max(1, depth - 1)):
            bk = hi_end + jnp.int32(k * r_chunk)

            @pl.when(bk < jnp.int32(nrows))
            def _(bk=bk):
                by_parity(bk, wait_half)


    return stage


_stage1 = _make_stage(
    nrows=320000, nseg_pad=64000, spw=2000, s_blk=200, r_chunk=256, depth=2
)
_stage2 = _make_stage(
    nrows=64000, nseg_pad=10240, spw=320, s_blk=160, r_chunk=256, depth=2
)


def kernel(x_main, x_mod, csr_idx_unit, csr_idx_view):
    del x_main
    offs1 = jnp.concatenate(
        [csr_idx_unit, jnp.broadcast_to(csr_idx_unit[-1:], (23,))]
    )
    offs2 = jnp.concatenate(
        [csr_idx_view, jnp.full((10256 - 10001,), 64000, jnp.int32)]
    )
    x_agg = _stage1(x_mod, offs1)
    out = _stage2(x_agg, offs2)
    return out[:10000]


# speedup vs baseline: 1.0630x; 1.0046x over previous
import functools
import jax
import jax.numpy as jnp
from jax import lax
from jax.experimental import pallas as pl
from jax.experimental.pallas import tpu as pltpu
from jax.experimental.pallas import tpu_sc as plsc

D = 128
LANES = 16
NV = D // LANES
NC = 2
NS = 16
NW = NC * NS


def _make_stage(nrows, nseg_pad, spw, s_blk, r_chunk, depth=2):
    assert nseg_pad == NW * spw and spw % s_blk == 0 and s_blk % 8 == 0
    assert nrows % r_chunk == 0 and (r_chunk & (r_chunk - 1)) == 0
    assert depth & (depth - 1) == 0
    nblk = spw // s_blk
    mesh = plsc.VectorSubcoreMesh(core_axis_name="c", subcore_axis_name="s")

    @functools.partial(
        pl.kernel,
        mesh=mesh,
        out_type=jax.ShapeDtypeStruct((nseg_pad, D), jnp.float32),
        scratch_types=[
            pltpu.VMEM((s_blk + LANES,), jnp.int32),
            pltpu.VMEM((depth * r_chunk, D), jnp.float32),
            pltpu.VMEM((2 * s_blk, D), jnp.float32),
            pltpu.SemaphoreType.DMA,
            pltpu.SemaphoreType.DMA,
        ]
        + [pltpu.SemaphoreType.DMA] * depth,
    )
    def stage(x_hbm, offs_hbm, out_hbm, offs_v, buf_v, out_v, sem, sem_o, *half_sems):
        wid = lax.axis_index("s") * NC + lax.axis_index("c")
        neg = jnp.full((LANES,), -jnp.inf, jnp.float32)
        zeros = jnp.zeros((LANES,), jnp.float32)
        cint = jnp.int32(r_chunk)

        def issue_half(base, p):
            base = pl.multiple_of(base, r_chunk)
            pltpu.async_copy(
                x_hbm.at[pl.ds(base, r_chunk)],
                buf_v.at[pl.ds(p * r_chunk, r_chunk)],
                half_sems[p],
            )

        def wait_half(p):
            pltpu.make_async_copy(
                x_hbm.at[pl.ds(0, r_chunk)],
                buf_v.at[pl.ds(p * r_chunk, r_chunk)],
                half_sems[p],
            ).wait()

        def by_parity(base, fn):
            slot = (base // cint) & (depth - 1)
            for p in range(depth):
                @pl.when(slot == p)
                def _(p=p):
                    fn(p)

        def refill(r):
            by_parity(r, wait_half)
            nb = r + jnp.int32((depth - 1) * r_chunk)

            @pl.when(nb < jnp.int32(nrows))
            def _():
                by_parity(nb, lambda p: issue_half(nb, p))

        pltpu.async_copy(
            offs_hbm.at[pl.ds(wid * spw, LANES)], offs_v.at[pl.ds(0, LANES)], sem
        ).wait()
        rbeg = offs_v[pl.ds(0, LANES)][0]
        b0 = jnp.minimum((rbeg // cint) * cint, jnp.int32(nrows - r_chunk))
        by_parity(b0, lambda p: issue_half(b0, p))
        by_parity(b0, wait_half)
        hi0 = b0 + cint
        for k in range(1, depth):
            bk = b0 + jnp.int32(k * r_chunk)

            @pl.when(bk < jnp.int32(nrows))
            def _(bk=bk):
                by_parity(bk, lambda p: issue_half(bk, p))

        ring_mask = jnp.int32(depth * r_chunk - 1)

        def blk_body(blk, carry):
            seg0 = wid * spw + blk * s_blk
            obase = jnp.bitwise_and(blk, 1) * jnp.int32(s_blk)
            pltpu.async_copy(offs_hbm.at[pl.ds(seg0, s_blk + LANES)], offs_v, sem).wait()

            @pl.when(blk > 0)
            def _():
                pltpu.make_async_copy(
                    out_v.at[pl.ds(0, s_blk)], out_hbm.at[pl.ds(0, s_blk)], sem_o
                ).wait()

            def seg_body(j, cst):
                hi_s, beg, end = cst
                end_nx = offs_v[pl.ds(j + 1, LANES)][1]

                ne = end > beg

                def store(acc):
                    for i in range(NV):
                        out_v[obase + j, pl.ds(i * LANES, LANES)] = jnp.where(
                            ne, acc[i], zeros
                        )

                def fast(_):
                    base = jnp.bitwise_and(beg, ring_mask)
                    lim = base + (end - beg)

                    @pl.loop(base, lim, init_carry=(neg,) * NV, step=2)
                    def facc(k, acc):
                        k2 = jnp.minimum(k + 1, lim - 1)
                        a = [
                            jnp.maximum(acc[i], buf_v[k, pl.ds(i * LANES, LANES)])
                            for i in range(NV)
                        ]
                        return tuple(
                            jnp.maximum(a[i], buf_v[k2, pl.ds(i * LANES, LANES)])
                            for i in range(NV)
                        )

                    store(facc)
                    return hi_s

                def slow(_):
                    def row_body(r, rst):
                        @pl.when(r >= rst[0])
                        def _():
                            refill(r)

                        hi = jnp.where(r >= rst[0], r + cint, rst[0])
                        row = jnp.bitwise_and(r, ring_mask)
                        acc = [
                            jnp.maximum(rst[1 + i], buf_v[row, pl.ds(i * LANES, LANES)])
                            for i in range(NV)
                        ]
                        return (hi, *acc)

                    rst = lax.fori_loop(beg, end, row_body, (hi_s,) + (neg,) * NV)
                    store(list(rst[1:]))
                    return rst[0]

                hi_n = lax.cond(end <= hi_s, fast, slow, 0)
                return (hi_n, end, end_nx)

            end0 = offs_v[pl.ds(0, LANES)][1]
            st = lax.fori_loop(
                0, s_blk, seg_body, (carry[0], carry[1], end0), unroll=4
            )
            carry = (st[0], st[1])
            pltpu.async_copy(
                out_v.at[pl.ds(obase, s_blk)], out_hbm.at[pl.ds(seg0, s_blk)], sem_o
            )
            return carry

        hi_end, _ = lax.fori_loop(0, nblk, blk_body, (hi0, rbeg))

        pltpu.make_async_copy(
            out_v.at[pl.ds(0, s_blk)], out_hbm.at[pl.ds(0, s_blk)], sem_o
        ).wait()

        for k in range(max(1, depth - 1)):
            bk = hi_end + jnp.int32(k * r_chunk)

            @pl.when(bk < jnp.int32(nrows))
            def _(bk=bk):
                by_parity(bk, wait_half)


    return stage


_stage1 = _make_stage(
    nrows=320000, nseg_pad=64000, spw=2000, s_blk=200, r_chunk=256, depth=2
)
_stage2 = _make_stage(
    nrows=64000, nseg_pad=10240, spw=320, s_blk=160, r_chunk=256, depth=2
)


def kernel(x_main, x_mod, csr_idx_unit, csr_idx_view):
    del x_main
    offs1 = jnp.concatenate(
        [csr_idx_unit, jnp.broadcast_to(csr_idx_unit[-1:], (23,))]
    )
    offs2 = jnp.concatenate(
        [csr_idx_view, jnp.full((10256 - 10001,), 64000, jnp.int32)]
    )
    x_agg = _stage1(x_mod, offs1)
    out = _stage2(x_agg, offs2)
    return out[:10000]
